# baseline (speedup 1.0000x reference)
import functools

import jax
import jax.numpy as jnp
from jax.experimental import pallas as pl
from jax.experimental.pallas import tpu as pltpu

_VMEM_LIMIT = 100 * 1024 * 1024


def _fused_body(x_ref, w1at_ref, w1t_ref, w2t_ref, o_ref):
    x = x_ref[...].astype(jnp.float32)
    sm = jnp.sum(x, axis=-1)
    mx = jnp.max(x, axis=-1)
    h = (jnp.maximum(jnp.dot(sm, w1at_ref[...],
                             preferred_element_type=jnp.float32), 0.0)
         + jnp.maximum(jnp.dot(mx, w1t_ref[...],
                               preferred_element_type=jnp.float32), 0.0))
    f = jnp.dot(h, w2t_ref[...], preferred_element_type=jnp.float32)
    attn = jax.nn.sigmoid(f)
    o_ref[...] = (x * attn[:, :, None]).astype(o_ref.dtype)


def kernel(x, w1, w2):
    n, c, h, w = x.shape
    cr = w1.shape[0]
    hw = h * w
    x_flat = x.reshape(n, c, hw)
    row_bytes = c * hw * jnp.dtype(x.dtype).itemsize

    budget = 8 * 1024 * 1024
    bt = 1
    for d in range(1, n + 1):
        if n % d == 0 and d * row_bytes <= budget and n // d >= 2:
            bt = d

    w1t = jnp.transpose(w1).astype(jnp.float32)
    w1at = w1t * jnp.float32(1.0 / hw)
    w2t = jnp.transpose(w2).astype(jnp.float32)

    out = pl.pallas_call(
        _fused_body,
        out_shape=jax.ShapeDtypeStruct((n, c, hw), x.dtype),
        grid=(n // bt,),
        in_specs=[
            pl.BlockSpec((bt, c, hw), lambda b: (b, 0, 0)),
            pl.BlockSpec((c, cr), lambda b: (0, 0)),
            pl.BlockSpec((c, cr), lambda b: (0, 0)),
            pl.BlockSpec((cr, c), lambda b: (0, 0)),
        ],
        out_specs=pl.BlockSpec((bt, c, hw), lambda b: (b, 0, 0)),
        compiler_params=pltpu.CompilerParams(
            dimension_semantics=("parallel",),
            vmem_limit_bytes=_VMEM_LIMIT,
        ),
    )(x_flat, w1at, w1t, w2t)
    return out.reshape(n, c, h, w)


# speedup vs baseline: 1.0075x; 1.0075x over previous
import jax
import jax.numpy as jnp
from jax.experimental import pallas as pl
from jax.experimental.pallas import tpu as pltpu

_VMEM_LIMIT = 100 * 1024 * 1024


def _make_fused_body(hw):
    inv_hw = 1.0 / float(hw)

    def _fused_body(x_ref, w1t_ref, w2t_ref, o_ref):
        x = x_ref[...].astype(jnp.float32)
        avg = jnp.sum(x, axis=-1) * inv_hw
        mx = jnp.max(x, axis=-1)
        w1t = w1t_ref[...]
        h = (jnp.maximum(jnp.dot(avg, w1t,
                                 preferred_element_type=jnp.float32), 0.0)
             + jnp.maximum(jnp.dot(mx, w1t,
                                   preferred_element_type=jnp.float32), 0.0))
        f = jnp.dot(h, w2t_ref[...], preferred_element_type=jnp.float32)
        attn = jax.nn.sigmoid(f)
        o_ref[...] = (x * attn[:, :, None]).astype(o_ref.dtype)

    return _fused_body


def kernel(x, w1, w2):
    n, c, h, w = x.shape
    cr = w1.shape[0]
    hw = h * w
    x_flat = x.reshape(n, c, hw)
    row_bytes = c * hw * jnp.dtype(x.dtype).itemsize

    budget = 8 * 1024 * 1024
    bt = 1
    for d in range(1, n + 1):
        if n % d == 0 and d * row_bytes <= budget and n // d >= 2:
            bt = d

    w1t = jnp.transpose(w1).astype(jnp.float32)
    w2t = jnp.transpose(w2).astype(jnp.float32)

    out = pl.pallas_call(
        _make_fused_body(hw),
        out_shape=jax.ShapeDtypeStruct((n, c, hw), x.dtype),
        grid=(n // bt,),
        in_specs=[
            pl.BlockSpec((bt, c, hw), lambda b: (b, 0, 0)),
            pl.BlockSpec((c, cr), lambda b: (0, 0)),
            pl.BlockSpec((cr, c), lambda b: (0, 0)),
        ],
        out_specs=pl.BlockSpec((bt, c, hw), lambda b: (b, 0, 0)),
        compiler_params=pltpu.CompilerParams(
            dimension_semantics=("parallel",),
            vmem_limit_bytes=_VMEM_LIMIT,
        ),
    )(x_flat, w1t, w2t)
    return out.reshape(n, c, h, w)
